# baseline (speedup 1.0000x reference)
import functools

import jax
import jax.numpy as jnp
from jax import lax
from jax.experimental import pallas as pl
from jax.experimental.pallas import tpu as pltpu

BST_DEPTH = 10
NB_NODES = 2**(BST_DEPTH + 1) - 1
LEVEL_OFF = [0, 1, 3, 7, 15, 31, 63, 128, 256, 512]


def _tree_half(xa, o_ref, r0, rows):
    i128 = lax.broadcasted_iota(jnp.int32, (rows, 128), 1)
    g128 = i128 // 2
    sgn128 = jnp.where(i128 % 2 == 0, 1.0, -1.0)

    c0 = xa[:, :128]
    par = jnp.maximum((i128 - 1) // 2, 0)
    sgn_odd = jnp.where(i128 % 2 == 1, 1.0, -1.0)
    e0 = jnp.where(i128 == 0, 1.0,
                   sgn_odd * jnp.take_along_axis(c0, par, axis=1))
    p2 = jnp.maximum((par - 1) // 2, 0)
    p4 = jnp.maximum((p2 - 1) // 2, 0)
    p4 = jnp.maximum((p4 - 1) // 2, 0)
    q0 = jnp.minimum(e0, jnp.take_along_axis(e0, par, axis=1))
    q0 = jnp.minimum(q0, jnp.take_along_axis(q0, p2, axis=1))
    q0 = jnp.minimum(q0, jnp.take_along_axis(q0, p4, axis=1))
    n1 = i128 + 128
    par1 = (n1 - 1) // 2
    sgn1 = jnp.where(n1 % 2 == 1, 1.0, -1.0)
    e1 = sgn1 * jnp.take_along_axis(c0, par1, axis=1)
    q1 = jnp.minimum(e1, jnp.take_along_axis(q0, par1, axis=1))
    o_ref[r0:r0 + rows, 0:128] = jnp.clip(q0, 0.0, 1.0)
    o_ref[r0:r0 + rows, 128:255] = jnp.clip(q1[:, :127], 0.0, 1.0)
    q0_127 = jnp.take_along_axis(q0, jnp.full((rows, 128), 127, jnp.int32),
                                 axis=1)
    p = [jnp.where(i128 == 0, q0_127, pltpu.roll(q1, 1, 1))]

    off = 255
    for d in range(7, BST_DEPTH):
        n = 2**(d + 1)
        o = LEVEL_OFF[d]
        chunks = []
        for j in range(n // 128):
            idx = g128 + 64 * (j % 2)
            src = xa[:, o + 128 * (j // 2):o + 128 * (j // 2) + 128]
            ssj = jnp.take_along_axis(src, idx, axis=1)
            spj = jnp.take_along_axis(p[j // 2], idx, axis=1)
            chunks.append(jnp.minimum(spj, ssj * sgn128))
        o_ref[r0:r0 + rows, off:off + n] = (
            jnp.clip(jnp.concatenate(chunks, axis=1), 0.0, 1.0))
        p = chunks
        off += n


def _lpsparse_kernel(x_ref, a_ref, o_ref, xa_scr, *, ntiles):
    i = pl.program_id(0)
    tb = x_ref.shape[0]

    xa_prev = xa_scr[pl.ds(((i - 1) % 2) * tb, tb), :]
    xa = lax.dot_general(
        x_ref[...], a_ref[...],
        dimension_numbers=(((1,), (1,)), ((), ())),
        preferred_element_type=jnp.float32,
    )
    xa_scr[pl.ds((i % 2) * tb, tb), :] = xa
    _tree_half(xa_prev, o_ref, 0, tb)


@functools.partial(jax.jit, static_argnames=("tb",))
def _run(x, a_pad, tb=1024):
    batch, dim = x.shape
    ntiles = batch // tb
    return pl.pallas_call(
        functools.partial(_lpsparse_kernel, ntiles=ntiles),
        grid=(ntiles + 1,),
        in_specs=[
            pl.BlockSpec((tb, dim), lambda i: (jnp.minimum(i, ntiles - 1), 0)),
            pl.BlockSpec((a_pad.shape[0], dim), lambda i: (0, 0)),
        ],
        out_specs=pl.BlockSpec((tb, NB_NODES),
                               lambda i: (jnp.maximum(i - 1, 0), 0)),
        out_shape=jax.ShapeDtypeStruct((batch, NB_NODES), jnp.float32),
        scratch_shapes=[pltpu.VMEM((2 * tb, 1024), jnp.float32)],
    )(x, a_pad)


def kernel(x, A):
    a_pad = jnp.concatenate(
        [A[:127], jnp.zeros((1, A.shape[1]), A.dtype), A[127:]], axis=0)
    return _run(x, a_pad)


# speedup vs baseline: 1.0249x; 1.0249x over previous
import functools

import jax
import jax.numpy as jnp
from jax import lax
from jax.experimental import pallas as pl
from jax.experimental.pallas import tpu as pltpu

BST_DEPTH = 10
NB_NODES = 2**(BST_DEPTH + 1) - 1
LEVEL_OFF = [0, 1, 3, 7, 15, 31, 63, 128, 256, 512]


def _tree_half(xa, o_ref, r0, rows):
    i128 = lax.broadcasted_iota(jnp.int32, (rows, 128), 1)
    g128 = i128 // 2
    sgn128 = jnp.where(i128 % 2 == 0, 1.0, -1.0)

    c0 = xa[:, :128]
    par = jnp.maximum((i128 - 1) // 2, 0)
    sgn_odd = jnp.where(i128 % 2 == 1, 1.0, -1.0)
    e0 = jnp.where(i128 == 0, 1.0,
                   sgn_odd * jnp.take_along_axis(c0, par, axis=1))
    p2 = jnp.maximum((par - 1) // 2, 0)
    p4 = jnp.maximum((p2 - 1) // 2, 0)
    p4 = jnp.maximum((p4 - 1) // 2, 0)
    q0 = jnp.minimum(e0, jnp.take_along_axis(e0, par, axis=1))
    q0 = jnp.minimum(q0, jnp.take_along_axis(q0, p2, axis=1))
    q0 = jnp.minimum(q0, jnp.take_along_axis(q0, p4, axis=1))
    n1 = i128 + 128
    par1 = (n1 - 1) // 2
    sgn1 = jnp.where(n1 % 2 == 1, 1.0, -1.0)
    e1 = sgn1 * jnp.take_along_axis(c0, par1, axis=1)
    q1 = jnp.minimum(e1, jnp.take_along_axis(q0, par1, axis=1))
    o_ref[r0:r0 + rows, 0:128] = jnp.clip(q0, 0.0, 1.0)
    o_ref[r0:r0 + rows, 128:255] = jnp.clip(q1[:, :127], 0.0, 1.0)
    q0_127 = jnp.take_along_axis(q0, jnp.full((rows, 128), 127, jnp.int32),
                                 axis=1)
    p = [jnp.where(i128 == 0, q0_127, pltpu.roll(q1, 1, 1))]

    off = 255
    for d in range(7, BST_DEPTH):
        n = 2**(d + 1)
        o = LEVEL_OFF[d]
        chunks = []
        for j in range(n // 128):
            idx = g128 + 64 * (j % 2)
            src = xa[:, o + 128 * (j // 2):o + 128 * (j // 2) + 128]
            ssj = jnp.take_along_axis(src, idx, axis=1)
            spj = jnp.take_along_axis(p[j // 2], idx, axis=1)
            chunks.append(jnp.minimum(spj, ssj * sgn128))
        o_ref[r0:r0 + rows, off:off + n] = (
            jnp.clip(jnp.concatenate(chunks, axis=1), 0.0, 1.0))
        p = chunks
        off += n


def _lpsparse_kernel(x_ref, a_ref, o_ref, xa_scr, *, ntiles):
    i = pl.program_id(0)
    tb = x_ref.shape[0]

    xa_prev = xa_scr[pl.ds(((i - 1) % 2) * tb, tb), :]
    xa = lax.dot_general(
        x_ref[...], a_ref[...],
        dimension_numbers=(((1,), (1,)), ((), ())),
        preferred_element_type=jnp.float32,
    )
    xa_scr[pl.ds((i % 2) * tb, tb), :] = xa
    _tree_half(xa_prev, o_ref, 0, tb)


@functools.partial(jax.jit, static_argnames=("tb",))
def _run(x, a_pad, tb=512):
    batch, dim = x.shape
    ntiles = batch // tb
    return pl.pallas_call(
        functools.partial(_lpsparse_kernel, ntiles=ntiles),
        grid=(ntiles + 1,),
        in_specs=[
            pl.BlockSpec((tb, dim), lambda i: (jnp.minimum(i, ntiles - 1), 0)),
            pl.BlockSpec((a_pad.shape[0], dim), lambda i: (0, 0)),
        ],
        out_specs=pl.BlockSpec((tb, NB_NODES),
                               lambda i: (jnp.maximum(i - 1, 0), 0)),
        out_shape=jax.ShapeDtypeStruct((batch, NB_NODES), jnp.float32),
        scratch_shapes=[pltpu.VMEM((2 * tb, 1024), jnp.float32)],
    )(x, a_pad)


def kernel(x, A):
    a_pad = jnp.concatenate(
        [A[:127], jnp.zeros((1, A.shape[1]), A.dtype), A[127:]], axis=0)
    return _run(x, a_pad)
